# baseline (speedup 1.0000x reference)
import functools

import jax
import jax.numpy as jnp
from jax import lax
from jax.experimental import pallas as pl
from jax.experimental.pallas import tpu as pltpu
from jax.experimental.pallas import tpu_sc as plsc


def _gather_body(num_chunks, chunk, uid_hbm, iid_hbm, ut_hbm, it_hbm,
                 out_u, out_i, uidx, iidx, urows, irows, sem_u, sem_i):
    nc = lax.axis_index("c")
    ns = lax.axis_index("s")
    wid = ns * 2 + nc
    b_per_w = num_chunks * chunk
    base = wid * b_per_w

    pltpu.sync_copy(uid_hbm.at[wid], uidx)
    pltpu.sync_copy(iid_hbm.at[wid], iidx)

    copies = []
    for j in range(num_chunks):
        c = pltpu.make_async_copy(ut_hbm.at[uidx.at[j]],
                                  urows.at[pl.ds(j * chunk, chunk)], sem_u)
        c.start()
        copies.append(c)
    for j in range(num_chunks):
        c = pltpu.make_async_copy(it_hbm.at[iidx.at[j]],
                                  irows.at[pl.ds(j * chunk, chunk)], sem_i)
        c.start()
        copies.append(c)
    for c in copies:
        c.wait()

    pltpu.sync_copy(urows, out_u.at[pl.ds(base, b_per_w)])
    pltpu.sync_copy(irows, out_i.at[pl.ds(base, b_per_w)])


def kernel(user_ids, item_ids, user_table, item_table):
    B = user_ids.shape[0]
    D = user_table.shape[1]
    info = plsc.get_sparse_core_info()
    nw = info.num_cores * info.num_subcores
    b_per_w = B // nw
    chunk = 128
    num_chunks = b_per_w // chunk

    uid3 = user_ids.astype(jnp.int32).reshape(nw, num_chunks, chunk)
    iid3 = item_ids.astype(jnp.int32).reshape(nw, num_chunks, chunk)

    mesh = plsc.VectorSubcoreMesh(core_axis_name="c", subcore_axis_name="s")
    out_sds = jax.ShapeDtypeStruct((B, D), jnp.float32)
    run = pl.kernel(
        functools.partial(_gather_body, num_chunks, chunk),
        out_type=(out_sds, out_sds),
        mesh=mesh,
        scratch_types=[
            pltpu.VMEM((num_chunks, chunk), jnp.int32),
            pltpu.VMEM((num_chunks, chunk), jnp.int32),
            pltpu.VMEM((b_per_w, D), jnp.float32),
            pltpu.VMEM((b_per_w, D), jnp.float32),
            pltpu.SemaphoreType.DMA,
            pltpu.SemaphoreType.DMA,
        ],
        compiler_params=pltpu.CompilerParams(use_tc_tiling_on_sc=False),
    )
    user_emb, item_emb = run(uid3, iid3, user_table, item_table)
    return (user_emb, user_emb, item_emb, item_emb)
